# baseline (speedup 1.0000x reference)
import jax
import jax.numpy as jnp
from jax.experimental import pallas as pl
from jax.experimental.pallas import tpu as pltpu

_H = 64


def kernel(interval, t, co_embeddings, divided, no_embeddings,
           unrelated_embeddings, is_last, hidden_state, W_ih, W_hh, b_ih,
           b_hh, W_t, b_t):
    N, G = co_embeddings.shape
    H = W_hh.shape[1]
    if hidden_state is None:
        hidden_state = jnp.zeros((N, H), co_embeddings.dtype)
    B = 12800
    grid = -(-N // B)

    interval_s = jnp.asarray(interval, jnp.float32).reshape(1)
    active_s = jnp.logical_not(is_last).astype(jnp.float32).reshape(1)

    xT = co_embeddings.T
    hT = hidden_state.T
    dT = divided.T
    wiT = W_ih.T
    whT = W_hh.T
    bi_row = b_ih[None, :]
    bh_row = b_hh[None, :]
    wt_row = W_t.T
    bt_row = b_t[None, :]

    def body(int_ref, act_ref, wi_ref, wh_ref, bi_ref, bh_ref, wt_ref,
             bt_ref, x_ref, h_ref, d_ref, out_ref, hnew_ref, a_ref, b_ref):
        i = pl.program_id(0)

        @pl.when(i == 0)
        def _init():
            out_ref[...] = jnp.full(out_ref.shape, -jnp.inf, jnp.float32)
            wi = wi_ref[...].astype(jnp.bfloat16)
            wh = wh_ref[...].astype(jnp.bfloat16)
            zb = jnp.zeros((H, H), jnp.bfloat16)
            a_ref[0:H, :] = jnp.concatenate([wi, zb], axis=1)
            a_ref[H:2 * H, 0:2 * H] = wh[:, 0:2 * H]
            a_ref[H:2 * H, 2 * H:3 * H] = zb
            a_ref[H:2 * H, 3 * H:4 * H] = wh[:, 2 * H:3 * H]
            bi = bi_ref[...]
            bh = bh_ref[...]
            stack = jnp.concatenate([bi + bh, bi, bh], axis=0)
            tr = jnp.transpose(stack, (1, 0))
            b_ref[0:2 * H, :] = tr[0:2 * H, 0:1]
            b_ref[2 * H:3 * H, :] = tr[2 * H:3 * H, 1:2]
            b_ref[3 * H:4 * H, :] = tr[2 * H:3 * H, 2:3]

        h = h_ref[...]
        xh = jnp.concatenate([x_ref[...].astype(jnp.bfloat16),
                              h.astype(jnp.bfloat16)], axis=0)
        g = jax.lax.dot_general(
            a_ref[...], xh, (((0,), (0,)), ((), ())),
            preferred_element_type=jnp.float32) + b_ref[...]
        rz = 0.5 * jnp.tanh(0.5 * g[0:2 * H]) + 0.5
        r = rz[0:H]
        z = rz[H:2 * H]
        n = jnp.tanh(g[2 * H:3 * H] + r * g[3 * H:4 * H])
        h_all = n + z * (h - n)

        dmax = jnp.max(d_ref[...], axis=0, keepdims=True)
        col = i * B + jax.lax.broadcasted_iota(jnp.int32, dmax.shape, 1)
        mask = (dmax > 0.0) & (act_ref[0] > 0.0) & (col < N)
        hnew_ref[...] = jnp.where(mask, h_all, 0.0)

        block_max = jnp.max(jnp.where(mask, h_all, -jnp.inf), axis=1,
                            keepdims=True)
        out_ref[...] = jnp.maximum(out_ref[...], block_max)

        @pl.when(i == grid - 1)
        def _finalize():
            inv = 1.0 / jnp.log(int_ref[0] + jnp.exp(1.0))
            wtbt = jnp.transpose(
                jnp.concatenate([wt_row_scale(wt_ref, inv), bt_ref[...]],
                                axis=0), (1, 0))
            tf = jnp.tanh(wtbt[:, 0:1] + wtbt[:, 1:2])
            out_ref[...] = out_ref[...] + tf

    def wt_row_scale(wt_ref, inv):
        return inv * wt_ref[...]

    out_col, h_newT = pl.pallas_call(
        body,
        grid=(grid,),
        in_specs=[
            pl.BlockSpec(memory_space=pltpu.SMEM),
            pl.BlockSpec(memory_space=pltpu.SMEM),
            pl.BlockSpec((G, 3 * H), lambda i: (0, 0)),
            pl.BlockSpec((H, 3 * H), lambda i: (0, 0)),
            pl.BlockSpec((1, 3 * H), lambda i: (0, 0)),
            pl.BlockSpec((1, 3 * H), lambda i: (0, 0)),
            pl.BlockSpec((1, H), lambda i: (0, 0)),
            pl.BlockSpec((1, H), lambda i: (0, 0)),
            pl.BlockSpec((G, B), lambda i: (0, i)),
            pl.BlockSpec((H, B), lambda i: (0, i)),
            pl.BlockSpec((3, B), lambda i: (0, i)),
        ],
        out_specs=[
            pl.BlockSpec((H, 1), lambda i: (0, 0)),
            pl.BlockSpec((H, B), lambda i: (0, i)),
        ],
        out_shape=[
            jax.ShapeDtypeStruct((H, 1), jnp.float32),
            jax.ShapeDtypeStruct((H, N), jnp.float32),
        ],
        scratch_shapes=[
            pltpu.VMEM((2 * H, 4 * H), jnp.bfloat16),
            pltpu.VMEM((4 * H, 1), jnp.float32),
        ],
        compiler_params=pltpu.CompilerParams(
            dimension_semantics=("arbitrary",)),
    )(interval_s, active_s, wiT, whT, bi_row, bh_row, wt_row, bt_row,
      xT, hT, dT)

    return (out_col[:, 0], h_newT.T)


# speedup vs baseline: 1.0116x; 1.0116x over previous
import jax
import jax.numpy as jnp
from jax.experimental import pallas as pl
from jax.experimental.pallas import tpu as pltpu

_H = 64


def kernel(interval, t, co_embeddings, divided, no_embeddings,
           unrelated_embeddings, is_last, hidden_state, W_ih, W_hh, b_ih,
           b_hh, W_t, b_t):
    N, G = co_embeddings.shape
    H = W_hh.shape[1]
    if hidden_state is None:
        hidden_state = jnp.zeros((N, H), co_embeddings.dtype)
    B = 14336
    grid = -(-N // B)

    interval_s = jnp.asarray(interval, jnp.float32).reshape(1)
    active_s = jnp.logical_not(is_last).astype(jnp.float32).reshape(1)

    xT = co_embeddings.T
    hT = hidden_state.T
    dT = divided.T
    wiT = W_ih.T
    whT = W_hh.T
    bi_row = b_ih[None, :]
    bh_row = b_hh[None, :]
    wt_row = W_t.T
    bt_row = b_t[None, :]

    def body(int_ref, act_ref, wi_ref, wh_ref, bi_ref, bh_ref, wt_ref,
             bt_ref, x_ref, h_ref, d_ref, out_ref, hnew_ref, a_ref, b_ref):
        i = pl.program_id(0)

        @pl.when(i == 0)
        def _init():
            out_ref[...] = jnp.full(out_ref.shape, -jnp.inf, jnp.float32)
            wi = wi_ref[...].astype(jnp.bfloat16)
            wh = wh_ref[...].astype(jnp.bfloat16)
            zb = jnp.zeros((H, H), jnp.bfloat16)
            a_ref[0:H, :] = jnp.concatenate([wi, zb], axis=1)
            a_ref[H:2 * H, 0:2 * H] = wh[:, 0:2 * H]
            a_ref[H:2 * H, 2 * H:3 * H] = zb
            a_ref[H:2 * H, 3 * H:4 * H] = wh[:, 2 * H:3 * H]
            bi = bi_ref[...]
            bh = bh_ref[...]
            stack = jnp.concatenate([bi + bh, bi, bh], axis=0)
            tr = jnp.transpose(stack, (1, 0))
            b_ref[0:2 * H, :] = tr[0:2 * H, 0:1]
            b_ref[2 * H:3 * H, :] = tr[2 * H:3 * H, 1:2]
            b_ref[3 * H:4 * H, :] = tr[2 * H:3 * H, 2:3]

        h = h_ref[...]
        xh = jnp.concatenate([x_ref[...].astype(jnp.bfloat16),
                              h.astype(jnp.bfloat16)], axis=0)
        g = jax.lax.dot_general(
            a_ref[...], xh, (((0,), (0,)), ((), ())),
            preferred_element_type=jnp.float32) + b_ref[...]
        rz = 0.5 * jnp.tanh(0.5 * g[0:2 * H]) + 0.5
        r = rz[0:H]
        z = rz[H:2 * H]
        n = jnp.tanh(g[2 * H:3 * H] + r * g[3 * H:4 * H])
        h_all = n + z * (h - n)

        dmax = jnp.max(d_ref[...], axis=0, keepdims=True)
        col = i * B + jax.lax.broadcasted_iota(jnp.int32, dmax.shape, 1)
        mask = (dmax > 0.0) & (act_ref[0] > 0.0) & (col < N)
        hnew_ref[...] = jnp.where(mask, h_all, 0.0)

        block_max = jnp.max(jnp.where(mask, h_all, -jnp.inf), axis=1,
                            keepdims=True)
        out_ref[...] = jnp.maximum(out_ref[...], block_max)

        @pl.when(i == grid - 1)
        def _finalize():
            inv = 1.0 / jnp.log(int_ref[0] + jnp.exp(1.0))
            wtbt = jnp.transpose(
                jnp.concatenate([wt_row_scale(wt_ref, inv), bt_ref[...]],
                                axis=0), (1, 0))
            tf = jnp.tanh(wtbt[:, 0:1] + wtbt[:, 1:2])
            out_ref[...] = out_ref[...] + tf

    def wt_row_scale(wt_ref, inv):
        return inv * wt_ref[...]

    out_col, h_newT = pl.pallas_call(
        body,
        grid=(grid,),
        in_specs=[
            pl.BlockSpec(memory_space=pltpu.SMEM),
            pl.BlockSpec(memory_space=pltpu.SMEM),
            pl.BlockSpec((G, 3 * H), lambda i: (0, 0)),
            pl.BlockSpec((H, 3 * H), lambda i: (0, 0)),
            pl.BlockSpec((1, 3 * H), lambda i: (0, 0)),
            pl.BlockSpec((1, 3 * H), lambda i: (0, 0)),
            pl.BlockSpec((1, H), lambda i: (0, 0)),
            pl.BlockSpec((1, H), lambda i: (0, 0)),
            pl.BlockSpec((G, B), lambda i: (0, i)),
            pl.BlockSpec((H, B), lambda i: (0, i)),
            pl.BlockSpec((3, B), lambda i: (0, i)),
        ],
        out_specs=[
            pl.BlockSpec((H, 1), lambda i: (0, 0)),
            pl.BlockSpec((H, B), lambda i: (0, i)),
        ],
        out_shape=[
            jax.ShapeDtypeStruct((H, 1), jnp.float32),
            jax.ShapeDtypeStruct((H, N), jnp.float32),
        ],
        scratch_shapes=[
            pltpu.VMEM((2 * H, 4 * H), jnp.bfloat16),
            pltpu.VMEM((4 * H, 1), jnp.float32),
        ],
        compiler_params=pltpu.CompilerParams(
            dimension_semantics=("arbitrary",)),
    )(interval_s, active_s, wiT, whT, bi_row, bh_row, wt_row, bt_row,
      xT, hT, dT)

    return (out_col[:, 0], h_newT.T)
